# Initial kernel scaffold; baseline (speedup 1.0000x reference)
import jax
import jax.numpy as jnp
from jax.experimental import pallas as pl


def kernel(pred, target, weights):
    raise NotImplementedError("write your pallas kernel here")


# speedup vs baseline: 12.8740x; 12.8740x over previous
import functools

import jax
import jax.numpy as jnp
from jax import lax
from jax.experimental import pallas as pl
from jax.experimental.pallas import tpu as pltpu
from jax.experimental.pallas import tpu_sc as plsc

_LAMBDA = 10.0
_NC = 2
_NS = 16
_NW = _NC * _NS
_L = 16


def _make_sc_loss(B, V, SPAD, R):
    assert B % (_NW * R) == 0 and V % 8 == 0
    RPW = B // _NW
    NCHUNK = RPW // R
    NFULL = V // _L
    REM = V - NFULL * _L
    TAIL = V - _L
    NJ = SPAD // _L

    mesh = plsc.VectorSubcoreMesh(core_axis_name="c", subcore_axis_name="s")

    @functools.partial(
        pl.kernel,
        mesh=mesh,
        out_type=jax.ShapeDtypeStruct((_NW, _L), jnp.float32),
        compiler_params=pltpu.CompilerParams(needs_layout_passes=False),
        scratch_types=[
            pltpu.VMEM((R, V), jnp.float32),
            pltpu.VMEM((R, SPAD), jnp.int32),
            pltpu.VMEM((RPW + _L,), jnp.float32),
            pltpu.VMEM((V + 8,), jnp.int32),
            pltpu.VMEM((_L,), jnp.float32),
        ],
    )
    def sc_loss(pred_hbm, tgt_hbm, w_hbm, out_hbm, pred_v, tgt_v, w_v, mark_v, out_v):
        wid = lax.axis_index("s") * _NC + lax.axis_index("c")
        row0 = wid * RPW
        lanes = lax.iota(jnp.int32, _L)
        tailf = jnp.where(lanes >= (_L - REM), 1.0, 0.0) if REM else jnp.zeros((_L,), jnp.float32)

        neg1 = jnp.full((_L,), -1, jnp.int32)

        def init_body(k, _):
            mark_v[pl.ds(k * _L, _L)] = neg1
            return 0

        lax.fori_loop(0, (V + 8) // _L, init_body, 0)

        pltpu.sync_copy(w_hbm.at[pl.ds(row0, RPW)], w_v.at[pl.ds(0, RPW)])

        def chunk_body(gidx, acc):
            base = row0 + gidx * R
            pltpu.sync_copy(pred_hbm.at[pl.ds(base, R), :], pred_v)
            pltpu.sync_copy(tgt_hbm.at[pl.ds(base, R), :], tgt_v)

            def row_body(r, acc2):
                mse_a, set_a = acc2

                def slice_body(k, dacc):
                    s2v, sev = dacc
                    p = pred_v[r, pl.ds(k * _L, _L)]
                    return (s2v + p * p, sev + jnp.exp(p))

                zero = jnp.zeros((_L,), jnp.float32)
                s2v, sev = lax.fori_loop(0, NFULL, slice_body, (zero, zero))
                pt = pred_v[r, pl.ds(TAIL, _L)]
                s2v = s2v + tailf * pt * pt
                sev = sev + tailf * jnp.exp(pt)
                s2 = jnp.sum(s2v)
                se = jnp.sum(sev)

                mbase = (gidx * R + r) * SPAD
                for j in range(NJ):
                    t = tgt_v[r, pl.ds(j * _L, _L)]
                    plsc.store_scatter(mark_v, [t], mbase + j * _L + lanes)
                npos_v = zero
                sg_v = zero
                spe_v = zero
                sne_v = zero
                rfull = jnp.full((_L,), r, jnp.int32)
                for j in range(NJ):
                    t = tgt_v[r, pl.ds(j * _L, _L)]
                    rb = plsc.load_gather(mark_v, [t])
                    keep = jnp.where(rb == mbase + j * _L + lanes, 1.0, 0.0)
                    g = plsc.load_gather(pred_v, [rfull, t])
                    npos_v = npos_v + keep
                    sg_v = sg_v + keep * g
                    spe_v = spe_v + keep * jnp.exp(-g)
                    sne_v = sne_v + keep * jnp.exp(g)
                bv = lambda x: jnp.full((_L,), x, jnp.float32)
                npos = bv(jnp.sum(npos_v))
                sg = bv(jnp.sum(sg_v))
                spe = bv(jnp.sum(spe_v))
                sne = bv(jnp.sum(sne_v))
                w = bv(w_v[pl.ds(gidx * R + r, _L)][0])
                mse_r = w * (bv(s2) - 2.0 * sg + npos)
                set_r = spe * (bv(se) - sne) / (npos * (float(V) - npos))
                return (mse_a + mse_r, set_a + set_r)

            return lax.fori_loop(0, R, row_body, acc)

        zv = jnp.zeros((_L,), jnp.float32)
        mse_acc, set_acc = lax.fori_loop(0, NCHUNK, chunk_body, (zv, zv))
        total = mse_acc * (1.0 / B) + (_LAMBDA / B) * set_acc
        out_v[...] = jnp.where(lanes == 0, total, 0.0)
        pltpu.sync_copy(out_v, out_hbm.at[wid])

    return sc_loss


def kernel(pred, target, weights):
    B, V = pred.shape
    S = target.shape[1]
    SPAD = 64
    tgt = jnp.concatenate(
        [target, jnp.broadcast_to(target[:, :1], (B, SPAD - S))], axis=1
    )
    partials = _make_sc_loss(B, V, SPAD, R=32)(pred, tgt, weights)
    return jnp.sum(partials)


# speedup vs baseline: 19.8088x; 1.5387x over previous
import functools

import jax
import jax.numpy as jnp
from jax import lax
from jax.experimental import pallas as pl
from jax.experimental.pallas import tpu as pltpu
from jax.experimental.pallas import tpu_sc as plsc

_LAMBDA = 10.0
_NC = 2
_NS = 16
_NW = _NC * _NS
_L = 16


def _make_sc_loss(B, V, SPAD, R):
    assert B % (_NW * R) == 0 and V % 8 == 0
    RPW = B // _NW
    NCHUNK = RPW // R
    NFULL = V // _L
    REM = V - NFULL * _L
    TAIL = V - _L
    NJ = SPAD // _L

    mesh = plsc.VectorSubcoreMesh(core_axis_name="c", subcore_axis_name="s")

    @functools.partial(
        pl.kernel,
        mesh=mesh,
        out_type=jax.ShapeDtypeStruct((_NW, _L), jnp.float32),
        compiler_params=pltpu.CompilerParams(needs_layout_passes=False),
        scratch_types=[
            pltpu.VMEM((R, V), jnp.float32),
            pltpu.VMEM((R, SPAD), jnp.int32),
            pltpu.VMEM((RPW + _L,), jnp.float32),
            pltpu.VMEM((V + 8,), jnp.int32),
            pltpu.VMEM((_L,), jnp.float32),
        ],
    )
    def sc_loss(pred_hbm, tgt_hbm, w_hbm, out_hbm, pred_v, tgt_v, w_v, mark_v, out_v):
        wid = lax.axis_index("s") * _NC + lax.axis_index("c")
        row0 = wid * RPW
        lanes = lax.iota(jnp.int32, _L)
        tailf = jnp.where(lanes >= (_L - REM), 1.0, 0.0) if REM else jnp.zeros((_L,), jnp.float32)

        neg1 = jnp.full((_L,), -1, jnp.int32)

        def init_body(k, _):
            mark_v[pl.ds(k * _L, _L)] = neg1
            return 0

        lax.fori_loop(0, (V + 8) // _L, init_body, 0)

        pltpu.sync_copy(w_hbm.at[pl.ds(row0, RPW)], w_v.at[pl.ds(0, RPW)])

        def chunk_body(gidx, acc):
            base = row0 + gidx * R
            pltpu.sync_copy(pred_hbm.at[pl.ds(base, R), :], pred_v)
            pltpu.sync_copy(tgt_hbm.at[pl.ds(base, R), :], tgt_v)

            def row_body(r, acc2):
                mse_a, set_a = acc2

                zero = jnp.zeros((_L,), jnp.float32)
                s2v = [zero, zero]
                sev = [zero, zero]
                for k in range(NFULL):
                    p = pred_v[r, pl.ds(k * _L, _L)]
                    s2v[k % 2] = s2v[k % 2] + p * p
                    sev[k % 2] = sev[k % 2] + jnp.exp(p)
                pt = pred_v[r, pl.ds(TAIL, _L)]
                s2 = jnp.sum(s2v[0] + s2v[1] + tailf * pt * pt)
                se = jnp.sum(sev[0] + sev[1] + tailf * jnp.exp(pt))

                mbase = (gidx * R + r) * SPAD
                for j in range(NJ):
                    t = tgt_v[r, pl.ds(j * _L, _L)]
                    plsc.store_scatter(mark_v, [t], mbase + j * _L + lanes)
                npos_v = zero
                sg_v = zero
                spe_v = zero
                sne_v = zero
                rfull = jnp.full((_L,), r, jnp.int32)
                for j in range(NJ):
                    t = tgt_v[r, pl.ds(j * _L, _L)]
                    rb = plsc.load_gather(mark_v, [t])
                    keep = jnp.where(rb == mbase + j * _L + lanes, 1.0, 0.0)
                    g = plsc.load_gather(pred_v, [rfull, t])
                    npos_v = npos_v + keep
                    sg_v = sg_v + keep * g
                    spe_v = spe_v + keep * jnp.exp(-g)
                    sne_v = sne_v + keep * jnp.exp(g)
                bv = lambda x: jnp.full((_L,), x, jnp.float32)
                npos = bv(jnp.sum(npos_v))
                sg = bv(jnp.sum(sg_v))
                spe = bv(jnp.sum(spe_v))
                sne = bv(jnp.sum(sne_v))
                w = bv(w_v[pl.ds(gidx * R + r, _L)][0])
                mse_r = w * (bv(s2) - 2.0 * sg + npos)
                set_r = spe * (bv(se) - sne) / (npos * (float(V) - npos))
                return (mse_a + mse_r, set_a + set_r)

            return lax.fori_loop(0, R, row_body, acc)

        zv = jnp.zeros((_L,), jnp.float32)
        mse_acc, set_acc = lax.fori_loop(0, NCHUNK, chunk_body, (zv, zv))
        total = mse_acc * (1.0 / B) + (_LAMBDA / B) * set_acc
        out_v[...] = jnp.where(lanes == 0, total, 0.0)
        pltpu.sync_copy(out_v, out_hbm.at[wid])

    return sc_loss


def kernel(pred, target, weights):
    B, V = pred.shape
    S = target.shape[1]
    SPAD = 64
    tgt = jnp.concatenate(
        [target, jnp.broadcast_to(target[:, :1], (B, SPAD - S))], axis=1
    )
    partials = _make_sc_loss(B, V, SPAD, R=32)(pred, tgt, weights)
    return jnp.sum(partials)
